# baseline (speedup 1.0000x reference)
import functools

import jax
import jax.numpy as jnp
from jax import lax
from jax.experimental import pallas as pl
from jax.experimental.pallas import tpu as pltpu
from jax.experimental.pallas import tpu_sc as plsc

TOKENS = 2048
HID = 1024
FF = 4096
NE = 8
CAP = 640
CAP_PAD = 648
NROWS = NE * CAP_PAD
NPAIR = 2 * TOKENS

SC_CORES = 2
SC_SUBCORES = 16
NWORK = SC_CORES * SC_SUBCORES
PAIRS_PER_W = NPAIR // NWORK
TOKS_PER_W = TOKENS // NWORK
CHUNK = 64


def _routing_body(x_ref, gamma_ref, beta_ref, wgt_ref,
                  flat_ref, slots_ref, w_ref, aux_ref, counts_ref):
    x = x_ref[...]
    mu = jnp.mean(x, axis=1, keepdims=True)
    xc = x - mu
    var = jnp.mean(xc * xc, axis=1, keepdims=True)
    flat = xc / jnp.sqrt(var + 1e-5) * gamma_ref[...] + beta_ref[...]
    flat_ref[...] = flat

    logits = jnp.dot(flat, wgt_ref[...], preferred_element_type=jnp.float32)
    mx = jnp.max(logits, axis=1, keepdims=True)
    eg = jnp.exp(logits - mx)
    gates = eg / jnp.sum(eg, axis=1, keepdims=True)

    colid = lax.broadcasted_iota(jnp.int32, (TOKENS, NE), 1)
    g0 = jnp.max(gates, axis=1, keepdims=True)
    idx0 = jnp.min(jnp.where(gates >= g0, colid, NE), axis=1, keepdims=True)
    m0 = colid == idx0
    gates_m = jnp.where(m0, jnp.float32(-1e30), gates)
    g1 = jnp.max(gates_m, axis=1, keepdims=True)
    idx1 = jnp.min(jnp.where(gates_m >= g1, colid, NE), axis=1, keepdims=True)
    m1 = colid == idx1

    def cumsum_tokens(m):
        s = m.astype(jnp.float32)
        d = 1
        while d < TOKENS:
            z = jnp.zeros((d, NE), dtype=jnp.float32)
            s = s + jnp.concatenate([z, s[:TOKENS - d, :]], axis=0)
            d *= 2
        return s

    c0 = cumsum_tokens(m0)
    loc0 = c0 - 1.0
    kept0 = m0 & (loc0 < CAP)
    used0 = jnp.sum(kept0.astype(jnp.float32), axis=0, keepdims=True)
    c1 = cumsum_tokens(m1)
    loc1 = c1 - 1.0 + used0
    kept1 = m1 & (loc1 < CAP)
    used1 = jnp.sum(kept1.astype(jnp.float32), axis=0, keepdims=True)

    k0 = jnp.max(kept0.astype(jnp.float32), axis=1, keepdims=True)
    k1 = jnp.max(kept1.astype(jnp.float32), axis=1, keepdims=True)
    gate0 = g0 * k0
    gate1 = g1 * k1
    denom = jnp.maximum(gate0 + gate1, 1e-9)
    w0 = gate0 / denom
    w1 = gate1 / denom
    w_ref[...] = jnp.concatenate([w0, w1], axis=1)

    loc0_t = jnp.sum(jnp.where(kept0, loc0, 0.0), axis=1, keepdims=True)
    loc1_t = jnp.sum(jnp.where(kept1, loc1, 0.0), axis=1, keepdims=True)
    slot0 = idx0 * CAP_PAD + jnp.where(k0 > 0.0, loc0_t.astype(jnp.int32), CAP)
    slot1 = idx1 * CAP_PAD + jnp.where(k1 > 0.0, loc1_t.astype(jnp.int32), CAP)
    slots_ref[...] = jnp.concatenate([slot0, slot1], axis=1)

    me = jnp.mean(gates, axis=0, keepdims=True)
    ce = used0 / jnp.float32(TOKENS)
    aux_ref[...] = jnp.sum(me * ce, axis=1, keepdims=True) * jnp.float32(NE)
    counts_ref[...] = used0 + used1


def _routing_call(x2, gamma2, beta2, wgt):
    return pl.pallas_call(
        _routing_body,
        out_shape=(
            jax.ShapeDtypeStruct((TOKENS, HID), jnp.float32),
            jax.ShapeDtypeStruct((TOKENS, 2), jnp.int32),
            jax.ShapeDtypeStruct((TOKENS, 2), jnp.float32),
            jax.ShapeDtypeStruct((1, 1), jnp.float32),
            jax.ShapeDtypeStruct((1, NE), jnp.float32),
        ),
    )(x2, gamma2, beta2, wgt)


def _dispatch_body(flat_hbm, slots_hbm, xdisp_hbm, idx0_v, idx1_v, rows_v, sem):
    wid = lax.axis_index("s") * SC_CORES + lax.axis_index("c")
    base = pl.multiple_of(wid * TOKS_PER_W, TOKS_PER_W)
    pltpu.sync_copy(slots_hbm.at[pl.ds(base, TOKS_PER_W)], idx0_v)
    pltpu.sync_copy(slots_hbm.at[pl.ds(TOKENS + base, TOKS_PER_W)], idx1_v)
    pltpu.sync_copy(flat_hbm.at[pl.ds(base, TOKS_PER_W)], rows_v)
    c0 = pltpu.async_copy(rows_v, xdisp_hbm.at[idx0_v], sem)
    c1 = pltpu.async_copy(rows_v, xdisp_hbm.at[idx1_v], sem)
    c0.wait()
    c1.wait()


def _dispatch_call(flat, slots):
    mesh = plsc.VectorSubcoreMesh(core_axis_name="c", subcore_axis_name="s",
                                  num_cores=SC_CORES, num_subcores=SC_SUBCORES)
    fn = pl.kernel(
        _dispatch_body,
        mesh=mesh,
        out_type=jax.ShapeDtypeStruct((NROWS, HID), jnp.float32),
        scratch_types=[
            pltpu.VMEM((TOKS_PER_W,), jnp.int32),
            pltpu.VMEM((TOKS_PER_W,), jnp.int32),
            pltpu.VMEM((TOKS_PER_W, HID), jnp.float32),
            pltpu.SemaphoreType.DMA,
        ],
    )
    return fn(flat, slots)


FTS = 1024
FT = FF // FTS


def _ffn_body(x_ref, w1_ref, b1_ref, w2_ref, b2_ref, y_ref):
    f = pl.program_id(1)
    x = x_ref[...].astype(jnp.bfloat16)
    h = lax.dot_general(x, w1_ref[0].astype(jnp.bfloat16),
                        (((1,), (1,)), ((), ())),
                        preferred_element_type=jnp.float32)
    h = h + b1_ref[0]
    h = 0.5 * h * (1.0 + lax.erf(h * jnp.float32(0.7071067811865476)))
    part = lax.dot_general(h.astype(jnp.bfloat16), w2_ref[0].astype(jnp.bfloat16),
                           (((1,), (1,)), ((), ())),
                           preferred_element_type=jnp.float32)

    @pl.when(f == 0)
    def _():
        y_ref[...] = part + b2_ref[0]

    @pl.when(f != 0)
    def _():
        y_ref[...] = y_ref[...] + part


def _ffn_call(xdisp, W1, b1, W2, b2):
    return pl.pallas_call(
        _ffn_body,
        grid=(NE, FT),
        in_specs=[
            pl.BlockSpec((CAP_PAD, HID), lambda e, f: (e, 0)),
            pl.BlockSpec((1, FTS, HID), lambda e, f: (e, f, 0)),
            pl.BlockSpec((1, 1, FTS), lambda e, f: (e * FT + f, 0, 0)),
            pl.BlockSpec((1, HID, FTS), lambda e, f: (e, 0, f)),
            pl.BlockSpec((1, 1, HID), lambda e, f: (e, 0, 0)),
        ],
        out_specs=pl.BlockSpec((CAP_PAD, HID), lambda e, f: (e, 0)),
        out_shape=jax.ShapeDtypeStruct((NROWS, HID), jnp.float32),
    )(xdisp, W1, b1.reshape(NE * FT, 1, FTS), W2, b2.reshape(NE, 1, HID))


CC = 16
NCH = TOKS_PER_W // CC


def _gcomb_body(y_hbm, slots_hbm, x_hbm, w0_hbm, w1_hbm, out_hbm,
                idx0a, idx0b, idx1a, idx1b, y0a, y0b, y1a, y1b,
                x_v, w0_v, w1_v, sema, semb):
    wid = lax.axis_index("s") * SC_CORES + lax.axis_index("c")
    base = pl.multiple_of(wid * TOKS_PER_W, TOKS_PER_W)
    idx0 = [idx0a, idx0b]
    idx1 = [idx1a, idx1b]
    y0s = [y0a, y0b]
    y1s = [y1a, y1b]
    sems = [sema, semb]

    def issue(c):
        b = c % 2
        toff = pl.multiple_of(base + c * CC, CC)
        pltpu.sync_copy(slots_hbm.at[pl.ds(toff, CC)], idx0[b])
        pltpu.sync_copy(slots_hbm.at[pl.ds(TOKENS + toff, CC)], idx1[b])
        g0 = pltpu.async_copy(y_hbm.at[idx0[b]], y0s[b], sems[b])
        g1 = pltpu.async_copy(y_hbm.at[idx1[b]], y1s[b], sems[b])
        return g0, g1

    pend = issue(0)
    for c in range(NCH):
        nxt = issue(c + 1) if c + 1 < NCH else None
        toff = pl.multiple_of(base + c * CC, CC)
        pltpu.sync_copy(x_hbm.at[pl.ds(toff, CC)], x_v)
        pltpu.sync_copy(w0_hbm.at[pl.ds(toff, CC)], w0_v)
        pltpu.sync_copy(w1_hbm.at[pl.ds(toff, CC)], w1_v)
        pend[0].wait()
        pend[1].wait()
        b = c % 2
        y0v = y0s[b]
        y1v = y1s[b]
        w0all = w0_v[...]
        w1all = w1_v[...]

        gdn = lax.GatherDimensionNumbers(
            offset_dims=(), collapsed_slice_dims=(0,), start_index_map=(0,))

        def row(r, _):
            ridx = jnp.full((16, 1), r, dtype=jnp.int32)
            w0s = lax.gather(w0all, ridx, gdn, (1,),
                             mode=lax.GatherScatterMode.PROMISE_IN_BOUNDS)
            w1s = lax.gather(w1all, ridx, gdn, (1,),
                             mode=lax.GatherScatterMode.PROMISE_IN_BOUNDS)
            for j in range(HID // 16):
                sl = pl.ds(j * 16, 16)
                x_v[r, sl] = (x_v[r, sl] + y0v[r, sl] * w0s
                              + y1v[r, sl] * w1s)
            return 0

        lax.fori_loop(0, CC, row, 0)
        pltpu.sync_copy(x_v, out_hbm.at[pl.ds(toff, CC)])
        pend = nxt


def _gcomb_call(y, slots, x2, w0, w1):
    mesh = plsc.VectorSubcoreMesh(core_axis_name="c", subcore_axis_name="s",
                                  num_cores=SC_CORES, num_subcores=SC_SUBCORES)
    fn = pl.kernel(
        _gcomb_body,
        mesh=mesh,
        out_type=jax.ShapeDtypeStruct((TOKENS, HID), jnp.float32),
        scratch_types=[
            pltpu.VMEM((CC,), jnp.int32),
            pltpu.VMEM((CC,), jnp.int32),
            pltpu.VMEM((CC,), jnp.int32),
            pltpu.VMEM((CC,), jnp.int32),
            pltpu.VMEM((CC, HID), jnp.float32),
            pltpu.VMEM((CC, HID), jnp.float32),
            pltpu.VMEM((CC, HID), jnp.float32),
            pltpu.VMEM((CC, HID), jnp.float32),
            pltpu.VMEM((CC, HID), jnp.float32),
            pltpu.VMEM((CC,), jnp.float32),
            pltpu.VMEM((CC,), jnp.float32),
            pltpu.SemaphoreType.DMA,
            pltpu.SemaphoreType.DMA,
        ],
    )
    return fn(y, slots, x2, w0, w1)


@jax.jit
def kernel(x, gamma, beta, wg, W1, b1, W2, b2):
    x2 = x.reshape(TOKENS, HID)
    flat, slots2, w2d, aux, counts = _routing_call(
        x2, gamma.reshape(1, HID), beta.reshape(1, HID), wg.T)
    slots = jnp.concatenate([slots2[:, 0], slots2[:, 1]])
    xdisp = _dispatch_call(flat, slots)
    y = _ffn_call(xdisp, W1, b1, W2, b2)
    out = _gcomb_call(y, slots, x2, w2d[:, 0], w2d[:, 1])
    return out.reshape(x.shape), aux[0, 0], counts[0]


# speedup vs baseline: 1.0774x; 1.0774x over previous
import functools

import jax
import jax.numpy as jnp
from jax import lax
from jax.experimental import pallas as pl
from jax.experimental.pallas import tpu as pltpu
from jax.experimental.pallas import tpu_sc as plsc

TOKENS = 2048
HID = 1024
FF = 4096
NE = 8
CAP = 640
CAP_PAD = 648
NROWS = NE * CAP_PAD
NPAIR = 2 * TOKENS

SC_CORES = 2
SC_SUBCORES = 16
NWORK = SC_CORES * SC_SUBCORES
PAIRS_PER_W = NPAIR // NWORK
TOKS_PER_W = TOKENS // NWORK
CHUNK = 64


def _routing_body(x_ref, gamma_ref, beta_ref, wgt_ref,
                  flat_ref, slots_ref, wexp0_ref, wexp1_ref, aux_ref,
                  counts_ref):
    x = x_ref[...]
    mu = jnp.mean(x, axis=1, keepdims=True)
    xc = x - mu
    var = jnp.mean(xc * xc, axis=1, keepdims=True)
    flat = xc / jnp.sqrt(var + 1e-5) * gamma_ref[...] + beta_ref[...]
    flat_ref[...] = flat

    logits = jnp.dot(flat, wgt_ref[...], preferred_element_type=jnp.float32)
    mx = jnp.max(logits, axis=1, keepdims=True)
    eg = jnp.exp(logits - mx)
    gates = eg / jnp.sum(eg, axis=1, keepdims=True)

    colid = lax.broadcasted_iota(jnp.int32, (TOKENS, NE), 1)
    g0 = jnp.max(gates, axis=1, keepdims=True)
    idx0 = jnp.min(jnp.where(gates >= g0, colid, NE), axis=1, keepdims=True)
    m0 = colid == idx0
    gates_m = jnp.where(m0, jnp.float32(-1e30), gates)
    g1 = jnp.max(gates_m, axis=1, keepdims=True)
    idx1 = jnp.min(jnp.where(gates_m >= g1, colid, NE), axis=1, keepdims=True)
    m1 = colid == idx1

    def cumsum_tokens(m):
        s = m.astype(jnp.float32)
        d = 1
        while d < TOKENS:
            z = jnp.zeros((d, NE), dtype=jnp.float32)
            s = s + jnp.concatenate([z, s[:TOKENS - d, :]], axis=0)
            d *= 2
        return s

    c0 = cumsum_tokens(m0)
    loc0 = c0 - 1.0
    kept0 = m0 & (loc0 < CAP)
    used0 = jnp.sum(kept0.astype(jnp.float32), axis=0, keepdims=True)
    c1 = cumsum_tokens(m1)
    loc1 = c1 - 1.0 + used0
    kept1 = m1 & (loc1 < CAP)
    used1 = jnp.sum(kept1.astype(jnp.float32), axis=0, keepdims=True)

    k0 = jnp.max(kept0.astype(jnp.float32), axis=1, keepdims=True)
    k1 = jnp.max(kept1.astype(jnp.float32), axis=1, keepdims=True)
    gate0 = g0 * k0
    gate1 = g1 * k1
    denom = jnp.maximum(gate0 + gate1, 1e-9)
    w0 = gate0 / denom
    w1 = gate1 / denom
    wexp0_ref[...] = jnp.broadcast_to(w0, (TOKENS, 128))
    wexp1_ref[...] = jnp.broadcast_to(w1, (TOKENS, 128))

    loc0_t = jnp.sum(jnp.where(kept0, loc0, 0.0), axis=1, keepdims=True)
    loc1_t = jnp.sum(jnp.where(kept1, loc1, 0.0), axis=1, keepdims=True)
    slot0 = idx0 * CAP_PAD + jnp.where(k0 > 0.0, loc0_t.astype(jnp.int32), CAP)
    slot1 = idx1 * CAP_PAD + jnp.where(k1 > 0.0, loc1_t.astype(jnp.int32), CAP)
    slots_ref[...] = jnp.concatenate([slot0, slot1], axis=1)

    me = jnp.mean(gates, axis=0, keepdims=True)
    ce = used0 / jnp.float32(TOKENS)
    aux_ref[...] = jnp.sum(me * ce, axis=1, keepdims=True) * jnp.float32(NE)
    counts_ref[...] = used0 + used1


def _routing_call(x2, gamma2, beta2, wgt):
    return pl.pallas_call(
        _routing_body,
        out_shape=(
            jax.ShapeDtypeStruct((TOKENS, HID), jnp.float32),
            jax.ShapeDtypeStruct((TOKENS, 2), jnp.int32),
            jax.ShapeDtypeStruct((TOKENS, 128), jnp.float32),
            jax.ShapeDtypeStruct((TOKENS, 128), jnp.float32),
            jax.ShapeDtypeStruct((1, 1), jnp.float32),
            jax.ShapeDtypeStruct((1, NE), jnp.float32),
        ),
    )(x2, gamma2, beta2, wgt)


def _dispatch_body(flat_hbm, slots_hbm, wexp0_hbm, wexp1_hbm,
                   xdisp_hbm, wslot_hbm,
                   idx0_v, idx1_v, rows_v, wexp0_v, wexp1_v, sem):
    wid = lax.axis_index("s") * SC_CORES + lax.axis_index("c")
    base = pl.multiple_of(wid * TOKS_PER_W, TOKS_PER_W)
    pltpu.sync_copy(slots_hbm.at[pl.ds(base, TOKS_PER_W)], idx0_v)
    pltpu.sync_copy(slots_hbm.at[pl.ds(TOKENS + base, TOKS_PER_W)], idx1_v)
    pltpu.sync_copy(flat_hbm.at[pl.ds(base, TOKS_PER_W)], rows_v)
    pltpu.sync_copy(wexp0_hbm.at[pl.ds(base, TOKS_PER_W)], wexp0_v)
    pltpu.sync_copy(wexp1_hbm.at[pl.ds(base, TOKS_PER_W)], wexp1_v)
    c0 = pltpu.async_copy(rows_v, xdisp_hbm.at[idx0_v], sem)
    c1 = pltpu.async_copy(rows_v, xdisp_hbm.at[idx1_v], sem)
    c2 = pltpu.async_copy(wexp0_v, wslot_hbm.at[idx0_v], sem)
    c3 = pltpu.async_copy(wexp1_v, wslot_hbm.at[idx1_v], sem)
    c0.wait()
    c1.wait()
    c2.wait()
    c3.wait()


def _dispatch_call(flat, slots, wexp0, wexp1):
    mesh = plsc.VectorSubcoreMesh(core_axis_name="c", subcore_axis_name="s",
                                  num_cores=SC_CORES, num_subcores=SC_SUBCORES)
    fn = pl.kernel(
        _dispatch_body,
        mesh=mesh,
        out_type=(
            jax.ShapeDtypeStruct((NROWS, HID), jnp.float32),
            jax.ShapeDtypeStruct((NROWS, 128), jnp.float32),
        ),
        scratch_types=[
            pltpu.VMEM((TOKS_PER_W,), jnp.int32),
            pltpu.VMEM((TOKS_PER_W,), jnp.int32),
            pltpu.VMEM((TOKS_PER_W, HID), jnp.float32),
            pltpu.VMEM((TOKS_PER_W, 128), jnp.float32),
            pltpu.VMEM((TOKS_PER_W, 128), jnp.float32),
            pltpu.SemaphoreType.DMA,
        ],
    )
    return fn(flat, slots, wexp0, wexp1)


FTS = 1024
FT = FF // FTS


def _ffn_body(x_ref, w1_ref, b1_ref, w2_ref, b2_ref, ws_ref, y_ref):
    f = pl.program_id(1)
    x = x_ref[...].astype(jnp.bfloat16)
    h = lax.dot_general(x, w1_ref[0].astype(jnp.bfloat16),
                        (((1,), (1,)), ((), ())),
                        preferred_element_type=jnp.float32)
    h = h + b1_ref[0]
    h = 0.5 * h * (1.0 + lax.erf(h * jnp.float32(0.7071067811865476)))
    part = lax.dot_general(h.astype(jnp.bfloat16), w2_ref[0].astype(jnp.bfloat16),
                           (((1,), (1,)), ((), ())),
                           preferred_element_type=jnp.float32)

    @pl.when(f == 0)
    def _():
        y_ref[...] = part + b2_ref[0]

    @pl.when(jnp.logical_and(f != 0, f != FT - 1))
    def _():
        y_ref[...] = y_ref[...] + part

    @pl.when(f == FT - 1)
    def _():
        y_ref[...] = (y_ref[...] + part) * ws_ref[:, 0:1]


def _ffn_call(xdisp, W1, b1, W2, b2, wslot):
    return pl.pallas_call(
        _ffn_body,
        grid=(NE, FT),
        in_specs=[
            pl.BlockSpec((CAP_PAD, HID), lambda e, f: (e, 0)),
            pl.BlockSpec((1, FTS, HID), lambda e, f: (e, f, 0)),
            pl.BlockSpec((1, 1, FTS), lambda e, f: (e * FT + f, 0, 0)),
            pl.BlockSpec((1, HID, FTS), lambda e, f: (e, 0, f)),
            pl.BlockSpec((1, 1, HID), lambda e, f: (e, 0, 0)),
            pl.BlockSpec((CAP_PAD, 128), lambda e, f: (e, 0)),
        ],
        out_specs=pl.BlockSpec((CAP_PAD, HID), lambda e, f: (e, 0)),
        out_shape=jax.ShapeDtypeStruct((NROWS, HID), jnp.float32),
    )(xdisp, W1, b1.reshape(NE * FT, 1, FTS), W2, b2.reshape(NE, 1, HID),
      wslot)


CC = 16
NCH = TOKS_PER_W // CC


def _gcomb_body(y_hbm, slots_hbm, x_hbm, out_hbm,
                idx0a, idx0b, idx1a, idx1b, xa, xb, sema, semb):
    wid = lax.axis_index("s") * SC_CORES + lax.axis_index("c")
    base = pl.multiple_of(wid * TOKS_PER_W, TOKS_PER_W)
    idx0 = [idx0a, idx0b]
    idx1 = [idx1a, idx1b]
    xs = [xa, xb]
    sems = [sema, semb]

    def load_and_issue(c):
        b = c % 2
        toff = pl.multiple_of(base + c * CC, CC)
        pltpu.sync_copy(slots_hbm.at[pl.ds(toff, CC)], idx0[b])
        pltpu.sync_copy(slots_hbm.at[pl.ds(TOKENS + toff, CC)], idx1[b])
        pltpu.sync_copy(x_hbm.at[pl.ds(toff, CC)], xs[b])
        g0 = pltpu.async_copy(y_hbm.at[idx0[b]], xs[b], sems[b], add=True)
        g1 = pltpu.async_copy(y_hbm.at[idx1[b]], xs[b], sems[b], add=True)
        return g0, g1

    pend = load_and_issue(0)
    for c in range(NCH):
        nxt = load_and_issue(c + 1) if c + 1 < NCH else None
        pend[0].wait()
        pend[1].wait()
        toff = pl.multiple_of(base + c * CC, CC)
        pltpu.sync_copy(xs[c % 2], out_hbm.at[pl.ds(toff, CC)])
        pend = nxt


def _gcomb_call(y, slots, x2):
    mesh = plsc.VectorSubcoreMesh(core_axis_name="c", subcore_axis_name="s",
                                  num_cores=SC_CORES, num_subcores=SC_SUBCORES)
    fn = pl.kernel(
        _gcomb_body,
        mesh=mesh,
        out_type=jax.ShapeDtypeStruct((TOKENS, HID), jnp.float32),
        scratch_types=[
            pltpu.VMEM((CC,), jnp.int32),
            pltpu.VMEM((CC,), jnp.int32),
            pltpu.VMEM((CC,), jnp.int32),
            pltpu.VMEM((CC,), jnp.int32),
            pltpu.VMEM((CC, HID), jnp.float32),
            pltpu.VMEM((CC, HID), jnp.float32),
            pltpu.SemaphoreType.DMA,
            pltpu.SemaphoreType.DMA,
        ],
    )
    return fn(y, slots, x2)


@jax.jit
def kernel(x, gamma, beta, wg, W1, b1, W2, b2):
    x2 = x.reshape(TOKENS, HID)
    flat, slots2, wexp0, wexp1, aux, counts = _routing_call(
        x2, gamma.reshape(1, HID), beta.reshape(1, HID), wg.T)
    slots = jnp.concatenate([slots2[:, 0], slots2[:, 1]])
    xdisp, wslot = _dispatch_call(flat, slots, wexp0, wexp1)
    y = _ffn_call(xdisp, W1, b1, W2, b2, wslot)
    out = _gcomb_call(y, slots, x2)
    return out.reshape(x.shape), aux[0, 0], counts[0]
